# baseline (device time: 51375 ns/iter reference)
import jax
import jax.numpy as jnp
from jax import lax
from jax.experimental import pallas as pl
from jax.experimental.pallas import tpu as pltpu

N_DEV = 4


def kernel(x, Wg, Wu, Wd):
    m, d = x.shape

    def body(x_ref, wg_ref, wu_ref, wd_ref, out_ref, comm_ref, send_sems, recv_sems):
        my_pos = lax.axis_index("i")
        left = lax.rem(my_pos + N_DEV - 1, N_DEV)
        right = lax.rem(my_pos + 1, N_DEV)

        barrier_sem = pltpu.get_barrier_semaphore()
        for nbr in [left, right]:
            pl.semaphore_signal(
                barrier_sem, inc=1,
                device_id=(nbr,), device_id_type=pl.DeviceIdType.MESH,
            )
        pl.semaphore_wait(barrier_sem, 2)

        gate = jnp.dot(x_ref[:, :], wg_ref[:, :], preferred_element_type=jnp.float32)
        up = jnp.dot(x_ref[:, :], wu_ref[:, :], preferred_element_type=jnp.float32)
        h = gate * (up * jax.nn.sigmoid(up))
        partial = jnp.dot(h, wd_ref[:, :], preferred_element_type=jnp.float32)

        out_ref[:, :] = partial
        comm_ref[0, :, :] = partial

        for k in range(N_DEV - 1):
            rdma = pltpu.make_async_remote_copy(
                src_ref=comm_ref.at[k],
                dst_ref=comm_ref.at[k + 1],
                send_sem=send_sems.at[k],
                recv_sem=recv_sems.at[k],
                device_id=(right,),
                device_id_type=pl.DeviceIdType.MESH,
            )
            rdma.start()
            rdma.wait()
            out_ref[:, :] += comm_ref[k + 1, :, :]

    return pl.pallas_call(
        body,
        out_shape=jax.ShapeDtypeStruct((m, d), jnp.float32),
        in_specs=[
            pl.BlockSpec(memory_space=pltpu.VMEM),
            pl.BlockSpec(memory_space=pltpu.VMEM),
            pl.BlockSpec(memory_space=pltpu.VMEM),
            pl.BlockSpec(memory_space=pltpu.VMEM),
        ],
        out_specs=pl.BlockSpec(memory_space=pltpu.VMEM),
        scratch_shapes=[
            pltpu.VMEM((N_DEV, m, d), jnp.float32),
            pltpu.SemaphoreType.DMA((N_DEV - 1,)),
            pltpu.SemaphoreType.DMA((N_DEV - 1,)),
        ],
        compiler_params=pltpu.CompilerParams(collective_id=0),
    )(x, Wg, Wu, Wd)


# device time: 26600 ns/iter; 1.9314x vs baseline; 1.9314x over previous
import jax
import jax.numpy as jnp
from jax import lax
from jax.experimental import pallas as pl
from jax.experimental.pallas import tpu as pltpu

N_DEV = 4


def kernel(x, Wg, Wu, Wd):
    m, d = x.shape
    mb = m // N_DEV

    def body(x_ref, wg_ref, wu_ref, wd_ref, out_ref,
             send_buf, rs_buf, rs_send_sems, rs_recv_sems,
             ag_send_sems, ag_recv_sems):
        me = lax.axis_index("i")

        barrier_sem = pltpu.get_barrier_semaphore()
        for r in range(1, N_DEV):
            pl.semaphore_signal(
                barrier_sem, inc=1,
                device_id=(lax.rem(me + r, N_DEV),),
                device_id_type=pl.DeviceIdType.MESH,
            )
        pl.semaphore_wait(barrier_sem, N_DEV - 1)

        def partial_block(t):
            xb = x_ref[pl.ds(t * mb, mb), :]
            gate = jnp.dot(xb, wg_ref[:, :], preferred_element_type=jnp.float32)
            up = jnp.dot(xb, wu_ref[:, :], preferred_element_type=jnp.float32)
            h = gate * (up * jax.nn.sigmoid(up))
            return jnp.dot(h, wd_ref[:, :], preferred_element_type=jnp.float32)

        for r in range(1, N_DEV):
            t = lax.rem(me + r, N_DEV)
            send_buf[r, :, :] = partial_block(t)
            rdma = pltpu.make_async_remote_copy(
                src_ref=send_buf.at[r],
                dst_ref=rs_buf.at[N_DEV - r],
                send_sem=rs_send_sems.at[r],
                recv_sem=rs_recv_sems.at[N_DEV - r],
                device_id=(t,),
                device_id_type=pl.DeviceIdType.MESH,
            )
            rdma.start()

        own = partial_block(me)

        for s in range(1, N_DEV):
            pltpu.make_async_remote_copy(
                src_ref=send_buf.at[s],
                dst_ref=rs_buf.at[s],
                send_sem=rs_send_sems.at[s],
                recv_sem=rs_recv_sems.at[s],
                device_id=(me,),
                device_id_type=pl.DeviceIdType.MESH,
            ).wait_recv()

        red = own + rs_buf[1, :, :] + rs_buf[2, :, :] + rs_buf[3, :, :]
        out_ref[pl.ds(me * mb, mb), :] = red

        for r in range(1, N_DEV):
            t = lax.rem(me + r, N_DEV)
            rdma = pltpu.make_async_remote_copy(
                src_ref=out_ref.at[pl.ds(me * mb, mb), :],
                dst_ref=out_ref.at[pl.ds(me * mb, mb), :],
                send_sem=ag_send_sems.at[r],
                recv_sem=ag_recv_sems.at[N_DEV - r],
                device_id=(t,),
                device_id_type=pl.DeviceIdType.MESH,
            )
            rdma.start()

        for s in range(1, N_DEV):
            u = lax.rem(me + s, N_DEV)
            pltpu.make_async_remote_copy(
                src_ref=out_ref.at[pl.ds(u * mb, mb), :],
                dst_ref=out_ref.at[pl.ds(u * mb, mb), :],
                send_sem=ag_send_sems.at[s],
                recv_sem=ag_recv_sems.at[s],
                device_id=(me,),
                device_id_type=pl.DeviceIdType.MESH,
            ).wait_recv()

        for r in range(1, N_DEV):
            t = lax.rem(me + r, N_DEV)
            pltpu.make_async_remote_copy(
                src_ref=send_buf.at[r],
                dst_ref=rs_buf.at[N_DEV - r],
                send_sem=rs_send_sems.at[r],
                recv_sem=rs_recv_sems.at[N_DEV - r],
                device_id=(t,),
                device_id_type=pl.DeviceIdType.MESH,
            ).wait_send()
            pltpu.make_async_remote_copy(
                src_ref=out_ref.at[pl.ds(me * mb, mb), :],
                dst_ref=out_ref.at[pl.ds(me * mb, mb), :],
                send_sem=ag_send_sems.at[r],
                recv_sem=ag_recv_sems.at[N_DEV - r],
                device_id=(t,),
                device_id_type=pl.DeviceIdType.MESH,
            ).wait_send()

    return pl.pallas_call(
        body,
        out_shape=jax.ShapeDtypeStruct((m, d), jnp.float32),
        in_specs=[
            pl.BlockSpec(memory_space=pltpu.VMEM),
            pl.BlockSpec(memory_space=pltpu.VMEM),
            pl.BlockSpec(memory_space=pltpu.VMEM),
            pl.BlockSpec(memory_space=pltpu.VMEM),
        ],
        out_specs=pl.BlockSpec(memory_space=pltpu.VMEM),
        scratch_shapes=[
            pltpu.VMEM((N_DEV, mb, d), jnp.float32),
            pltpu.VMEM((N_DEV, mb, d), jnp.float32),
            pltpu.SemaphoreType.DMA((N_DEV,)),
            pltpu.SemaphoreType.DMA((N_DEV,)),
            pltpu.SemaphoreType.DMA((N_DEV,)),
            pltpu.SemaphoreType.DMA((N_DEV,)),
        ],
        compiler_params=pltpu.CompilerParams(collective_id=0),
    )(x, Wg, Wu, Wd)
